# Initial kernel scaffold; baseline (speedup 1.0000x reference)
import functools

import jax
import jax.numpy as jnp
from jax import lax
from jax.experimental import pallas as pl
from jax.experimental.pallas import tpu as pltpu
from jax.experimental.pallas import tpu_sc as plsc

N_NODES = 10000
N_EDGES = 320000
D = 128

NC = 2
NS = 16
NW = NC * NS
E_PER_W = N_EDGES // NW
CHUNK = 80
N_CHUNKS = E_PER_W // CHUNK
ROWS_PER_S = N_NODES // NS


def _sc_aggregate(h, src, dst, zeros):
    mesh = plsc.VectorSubcoreMesh(core_axis_name="c", subcore_axis_name="s")

    @functools.partial(
        pl.kernel,
        out_type=jax.ShapeDtypeStruct((NC, N_NODES, D), jnp.float32),
        mesh=mesh,
        scratch_types=[
            pltpu.VMEM((N_CHUNKS, CHUNK), jnp.int32),
            pltpu.VMEM((N_CHUNKS, CHUNK), jnp.int32),
            pltpu.VMEM((CHUNK, D), jnp.float32),
            pltpu.VMEM_SHARED((N_NODES, D), jnp.float32),
            pltpu.SemaphoreType.DMA,
        ],
    )
    def agg(h_hbm, src_hbm, dst_hbm, zeros_hbm, out_hbm,
            src_v, dst_v, rows_v, acc, sem):
        c = lax.axis_index("c")
        s = lax.axis_index("s")
        wid = c * NS + s

        pltpu.sync_copy(zeros_hbm.at[pl.ds(s * ROWS_PER_S, ROWS_PER_S)],
                        acc.at[pl.ds(s * ROWS_PER_S, ROWS_PER_S)])
        plsc.subcore_barrier()

        pltpu.sync_copy(src_hbm.at[wid], src_v)
        pltpu.sync_copy(dst_hbm.at[wid], dst_v)

        def body(i, carry):
            pltpu.async_copy(h_hbm.at[src_v.at[i]], rows_v, sem).wait()
            pltpu.sync_copy(rows_v, acc.at[dst_v.at[i]], add=True)
            return carry

        lax.fori_loop(0, N_CHUNKS, body, 0)
        plsc.subcore_barrier()

        pltpu.sync_copy(acc.at[pl.ds(s * ROWS_PER_S, ROWS_PER_S)],
                        out_hbm.at[c, pl.ds(s * ROWS_PER_S, ROWS_PER_S)])

    return agg(h, src, dst, zeros)


def _tc_linear(p0, p1, W, b):
    BLK = 400
    grid = N_NODES // BLK

    def body(p0_ref, p1_ref, w_ref, b_ref, out_ref):
        ah = p0_ref[...] + p1_ref[...]
        out_ref[...] = jnp.maximum(
            jnp.dot(ah, w_ref[...], preferred_element_type=jnp.float32)
            + b_ref[...], 0.0)

    return pl.pallas_call(
        body,
        grid=(grid,),
        in_specs=[
            pl.BlockSpec((BLK, D), lambda i: (i, 0)),
            pl.BlockSpec((BLK, D), lambda i: (i, 0)),
            pl.BlockSpec((D, D), lambda i: (0, 0)),
            pl.BlockSpec((1, D), lambda i: (0, 0)),
        ],
        out_specs=pl.BlockSpec((BLK, D), lambda i: (i, 0)),
        out_shape=jax.ShapeDtypeStruct((N_NODES, D), jnp.float32),
    )(p0, p1, W, b)


def kernel(h, edge_index, W, b):
    ei = edge_index.astype(jnp.int32)
    src = ei[0].reshape(NW, N_CHUNKS, CHUNK)
    dst = ei[1].reshape(NW, N_CHUNKS, CHUNK)
    zeros = jnp.zeros((N_NODES, D), jnp.float32)
    partials = _sc_aggregate(h, src, dst, zeros)
    return _tc_linear(partials[0], partials[1], W, b.reshape(1, D))


# speedup vs baseline: 7.1989x; 7.1989x over previous
import functools

import jax
import jax.numpy as jnp
from jax import lax
from jax.experimental import pallas as pl
from jax.experimental.pallas import tpu as pltpu
from jax.experimental.pallas import tpu_sc as plsc

N_NODES = 10000
N_EDGES = 320000
D = 128

NC = 2
NS = 16
NW = NC * NS
E_PER_W = N_EDGES // NW
CHUNK = 80
N_CHUNKS = E_PER_W // CHUNK
N_PAD = 10240
ROWS_PER_S = N_PAD // NS


def _sc_aggregate(h, src, dst, zeros):
    mesh = plsc.VectorSubcoreMesh(core_axis_name="c", subcore_axis_name="s")

    @functools.partial(
        pl.kernel,
        out_type=jax.ShapeDtypeStruct((NC, N_PAD, D), jnp.float32),
        mesh=mesh,
        scratch_types=[
            pltpu.VMEM((N_CHUNKS, CHUNK), jnp.int32),
            pltpu.VMEM((N_CHUNKS, CHUNK), jnp.int32),
            pltpu.VMEM((CHUNK, D), jnp.float32),
            pltpu.VMEM_SHARED((N_PAD, D), jnp.float32),
            pltpu.SemaphoreType.DMA,
        ],
    )
    def agg(h_hbm, src_hbm, dst_hbm, zeros_hbm, out_hbm,
            src_v, dst_v, rows_v, acc, sem):
        c = lax.axis_index("c")
        s = lax.axis_index("s")
        wid = c * NS + s

        pltpu.sync_copy(zeros_hbm.at[pl.ds(s * ROWS_PER_S, ROWS_PER_S)],
                        acc.at[pl.ds(s * ROWS_PER_S, ROWS_PER_S)])
        plsc.subcore_barrier()

        pltpu.sync_copy(src_hbm.at[wid], src_v)
        pltpu.sync_copy(dst_hbm.at[wid], dst_v)

        def body(i, carry):
            pltpu.async_copy(h_hbm.at[src_v.at[i]], rows_v, sem).wait()
            pltpu.sync_copy(rows_v, acc.at[dst_v.at[i]], add=True)
            return carry

        lax.fori_loop(0, N_CHUNKS, body, 0)
        plsc.subcore_barrier()

        pltpu.sync_copy(acc.at[pl.ds(s * ROWS_PER_S, ROWS_PER_S)],
                        out_hbm.at[c, pl.ds(s * ROWS_PER_S, ROWS_PER_S)])

    return agg(h, src, dst, zeros)


def _tc_linear(p0, p1, W, b):
    BLK = 400
    grid = N_NODES // BLK

    def body(p0_ref, p1_ref, w_ref, b_ref, out_ref):
        ah = p0_ref[...] + p1_ref[...]
        out_ref[...] = jnp.maximum(
            jnp.dot(ah, w_ref[...], preferred_element_type=jnp.float32)
            + b_ref[...], 0.0)

    return pl.pallas_call(
        body,
        grid=(grid,),
        in_specs=[
            pl.BlockSpec((BLK, D), lambda i: (i, 0)),
            pl.BlockSpec((BLK, D), lambda i: (i, 0)),
            pl.BlockSpec((D, D), lambda i: (0, 0)),
            pl.BlockSpec((1, D), lambda i: (0, 0)),
        ],
        out_specs=pl.BlockSpec((BLK, D), lambda i: (i, 0)),
        out_shape=jax.ShapeDtypeStruct((N_NODES, D), jnp.float32),
    )(p0, p1, W, b)


def kernel(h, edge_index, W, b):
    ei = edge_index.astype(jnp.int32)
    src = ei[0].reshape(NW, N_CHUNKS, CHUNK)
    dst = ei[1].reshape(NW, N_CHUNKS, CHUNK)
    zeros = jnp.zeros((N_PAD, D), jnp.float32)
    partials = _sc_aggregate(h, src, dst, zeros)
    return _tc_linear(partials[0], partials[1], W, b.reshape(1, D))


# speedup vs baseline: 9.2003x; 1.2780x over previous
import functools

import jax
import jax.numpy as jnp
from jax import lax
from jax.experimental import pallas as pl
from jax.experimental.pallas import tpu as pltpu
from jax.experimental.pallas import tpu_sc as plsc

N_NODES = 10000
N_EDGES = 320000
D = 128

NC = 2
NS = 16
NW = NC * NS
E_PER_W = N_EDGES // NW
CHUNK = 80
N_CHUNKS = E_PER_W // CHUNK
N_PAD = 10112
ROWS_PER_S = N_PAD // NS


def _sc_aggregate(h, src, dst, zeros):
    mesh = plsc.VectorSubcoreMesh(core_axis_name="c", subcore_axis_name="s")

    @functools.partial(
        pl.kernel,
        out_type=jax.ShapeDtypeStruct((NC, N_PAD, D), jnp.float32),
        mesh=mesh,
        scratch_types=[
            pltpu.VMEM((E_PER_W,), jnp.int32),
            pltpu.VMEM((N_CHUNKS, CHUNK), jnp.int32),
            pltpu.VMEM((2, CHUNK, D), jnp.float32),
            pltpu.VMEM_SHARED((N_PAD, D), jnp.float32),
            pltpu.SemaphoreType.DMA,
        ],
    )
    def agg(h_hbm, src_hbm, dst_hbm, zeros_hbm, out_hbm,
            src_v, dst_v, rows_v, acc, sem):
        c = lax.axis_index("c")
        s = lax.axis_index("s")
        wid = c * NS + s

        pltpu.sync_copy(zeros_hbm.at[pl.ds(s * ROWS_PER_S, ROWS_PER_S)],
                        acc.at[pl.ds(s * ROWS_PER_S, ROWS_PER_S)])
        plsc.subcore_barrier()

        pltpu.sync_copy(src_hbm.at[wid], src_v)
        pltpu.sync_copy(dst_hbm.at[wid], dst_v)

        pltpu.async_copy(h_hbm.at[src_v.at[pl.ds(0, CHUNK)]],
                         rows_v.at[0], sem)

        def body(i, carry):
            buf = lax.rem(i, 2)
            pltpu.make_async_copy(h_hbm.at[src_v.at[pl.ds(i * CHUNK, CHUNK)]],
                                  rows_v.at[buf], sem).wait()

            @pl.when(i + 1 < N_CHUNKS)
            def _():
                pltpu.async_copy(
                    h_hbm.at[src_v.at[pl.ds((i + 1) * CHUNK, CHUNK)]],
                    rows_v.at[1 - buf], sem)

            pltpu.sync_copy(rows_v.at[buf], acc.at[dst_v.at[i]], add=True)
            return carry

        lax.fori_loop(0, N_CHUNKS, body, 0)
        plsc.subcore_barrier()

        pltpu.sync_copy(acc.at[pl.ds(s * ROWS_PER_S, ROWS_PER_S)],
                        out_hbm.at[c, pl.ds(s * ROWS_PER_S, ROWS_PER_S)])

    return agg(h, src, dst, zeros)


def _tc_linear(partials, W, b):
    BLK = 400
    grid = N_NODES // BLK

    def body(p0_ref, p1_ref, w_ref, b_ref, out_ref):
        ah = p0_ref[0] + p1_ref[0]
        out_ref[...] = jnp.maximum(
            jnp.dot(ah, w_ref[...], preferred_element_type=jnp.float32)
            + b_ref[...], 0.0)

    return pl.pallas_call(
        body,
        grid=(grid,),
        in_specs=[
            pl.BlockSpec((1, BLK, D), lambda i: (0, i, 0)),
            pl.BlockSpec((1, BLK, D), lambda i: (1, i, 0)),
            pl.BlockSpec((D, D), lambda i: (0, 0)),
            pl.BlockSpec((1, D), lambda i: (0, 0)),
        ],
        out_specs=pl.BlockSpec((BLK, D), lambda i: (i, 0)),
        out_shape=jax.ShapeDtypeStruct((N_NODES, D), jnp.float32),
    )(partials, partials, W, b)


def kernel(h, edge_index, W, b):
    ei = edge_index.astype(jnp.int32)
    src = ei[0].reshape(NW, E_PER_W)
    dst = ei[1].reshape(NW, N_CHUNKS, CHUNK)
    zeros = jnp.zeros((N_PAD, D), jnp.float32)
    partials = _sc_aggregate(h, src, dst, zeros)
    return _tc_linear(partials, W, b.reshape(1, D))
